# baseline (speedup 1.0000x reference)
import jax
import jax.numpy as jnp
import numpy as np
from jax import lax
from jax.experimental import pallas as pl
from jax.experimental.pallas import tpu as pltpu
from jax.experimental.pallas import tpu_sc as plsc

N_NODES = 100000
N_EDGES = 6400000
NUM_TYPES = 16
FACTOR = 1.0 / np.sqrt(64.0)

NC = 2
NS = 16
NW = NC * NS
L = 16

K = 512
NCH = N_EDGES // K
NT_MIN = NCH // NW
NT_REM = NCH - NT_MIN * NW
NBUF = 4

NSEG = 6256
NPAD = NS * NSEG


def _sc_body(energy_hbm, eidx_hbm, species_hbm, scale_hbm, out_hbm,
             species_v, scale_v,
             e0, e1, e2, e3, cn0, cn1, cn2, cn3, v0, v1, v2, v3,
             x0, x1, x2, x3, s0, s1, s2, s3, ss0, ss1, zbuf, accum_sh):
    cid = lax.axis_index("c")
    sid = lax.axis_index("s")
    wid = cid * NS + sid

    e_b = (e0, e1, e2, e3)
    cn_b = (cn0, cn1, cn2, cn3)
    v_b = (v0, v1, v2, v3)
    x_b = (x0, x1, x2, x3)
    sem_b = (s0, s1, s2, s3)
    sem_s = (ss0, ss1)

    pltpu.sync_copy(species_hbm, species_v)
    pltpu.sync_copy(scale_hbm, scale_v)

    def zbody(i, _):
        zbuf[pl.ds(i * L, L)] = jnp.zeros((L,), jnp.float32)
        return _

    lax.fori_loop(0, NSEG // L, zbody, None)
    pltpu.sync_copy(zbuf, accum_sh.at[pl.ds(sid * NSEG, NSEG)])
    plsc.subcore_barrier()

    def base_of(t):
        return (t * NW + wid) * K

    def fire_in(t, b):
        base = base_of(t)
        pltpu.async_copy(energy_hbm.at[pl.ds(base, K)], e_b[b], sem_b[b])
        pltpu.async_copy(eidx_hbm.at[:, pl.ds(base, K)], cn_b[b], sem_b[b])

    def wait_in(t, b):
        base = base_of(t)
        pltpu.make_async_copy(energy_hbm.at[pl.ds(base, K)], e_b[b],
                              sem_b[b]).wait()
        pltpu.make_async_copy(eidx_hbm.at[:, pl.ds(base, K)], cn_b[b],
                              sem_b[b]).wait()

    def compute(t, b):
        @plsc.parallel_loop(0, K, step=L, unroll=4)
        def gbody(off):
            ci = cn_b[b][0, pl.ds(off, L)]
            ni = cn_b[b][1, pl.ds(off, L)]
            x_b[b][pl.ds(off, L)] = ci
            sc = plsc.load_gather(species_v, [ci])
            sn = plsc.load_gather(species_v, [ni])
            comb = (sc << 4) + sn
            v_b[b][pl.ds(off, L)] = e_b[b][pl.ds(off, L)] * \
                plsc.load_gather(scale_v, [comb])

    def fire_scatter(b, p):
        pltpu.async_copy(v_b[b], accum_sh.at[x_b[b]], sem_s[p], add=True)

    def wait_scatter(b, p):
        pltpu.make_async_copy(v_b[b], accum_sh.at[x_b[b]], sem_s[p]).wait()

    def step(t, j):
        wait_in(t, j)
        wait_scatter((j + 2) % NBUF, j % 2)
        fire_in(t + 2, (j + 2) % NBUF)
        compute(t, j)
        fire_scatter(j, j % 2)

    fire_in(0, 0)
    fire_in(1, 1)
    for t in range(NBUF):
        wait_in(t, t % NBUF)
        if t >= 2:
            wait_scatter((t + 2) % NBUF, t % 2)
        fire_in(t + 2, (t + 2) % NBUF)
        compute(t, t % NBUF)
        fire_scatter(t % NBUF, t % 2)

    def block_body(t4, _):
        for j in range(NBUF):
            step(t4 * NBUF + j, j)
        return _

    lax.fori_loop(1, (NT_MIN - NBUF) // NBUF, block_body, None)

    extra = wid < NT_REM

    for t in range((NT_MIN - NBUF) // NBUF * NBUF, NT_MIN):
        j = t % NBUF
        wait_in(t, j)
        wait_scatter((j + 2) % NBUF, j % 2)
        if t + 2 < NT_MIN:
            fire_in(t + 2, (j + 2) % NBUF)
        elif t + 2 == NT_MIN:
            @pl.when(extra)
            def _fire_extra(t=t, j=j):
                fire_in(t + 2, (j + 2) % NBUF)
        compute(t, j)
        fire_scatter(j, j % 2)
    for t in (NT_MIN - 2, NT_MIN - 1):
        wait_scatter(t % NBUF, t % 2)

    @pl.when(extra)
    def _extra_chunk():
        j = NT_MIN % NBUF
        wait_in(NT_MIN, j)
        compute(NT_MIN, j)
        pltpu.sync_copy(v_b[j], accum_sh.at[x_b[j]], add=True)

    plsc.subcore_barrier()

    pltpu.sync_copy(accum_sh.at[pl.ds(sid * NSEG, NSEG)], zbuf)
    pltpu.sync_copy(zbuf, out_hbm.at[pl.ds(cid * NPAD + sid * NSEG, NSEG)])


@jax.jit
def _sc_call(energy, eidx, species, scale):
    mesh = plsc.VectorSubcoreMesh(core_axis_name="c", subcore_axis_name="s")
    return pl.kernel(
        _sc_body,
        out_type=jax.ShapeDtypeStruct((NC * NPAD,), jnp.float32),
        mesh=mesh,
        compiler_params=pltpu.CompilerParams(needs_layout_passes=False),
        scratch_types=[
            pltpu.VMEM((N_NODES,), jnp.int32),
            pltpu.VMEM((NUM_TYPES * NUM_TYPES,), jnp.float32),
            *([pltpu.VMEM((K,), jnp.float32)] * NBUF),
            *([pltpu.VMEM((2, K), jnp.int32)] * NBUF),
            *([pltpu.VMEM((K,), jnp.float32)] * NBUF),
            *([pltpu.VMEM((K,), jnp.int32)] * NBUF),
            *([pltpu.SemaphoreType.DMA] * NBUF),
            pltpu.SemaphoreType.DMA,
            pltpu.SemaphoreType.DMA,
            pltpu.VMEM((NSEG,), jnp.float32),
            pltpu.VMEM_SHARED((NPAD,), jnp.float32),
        ],
    )(energy, eidx, species, scale)


def kernel(edge_energy, per_edge_scales, edge_index, atom_types):
    energy = edge_energy.reshape(N_EDGES)
    scale = (per_edge_scales * FACTOR).reshape(NUM_TYPES * NUM_TYPES)
    species = atom_types.reshape(N_NODES)
    partials = _sc_call(energy, edge_index, species, scale)
    return (partials[:N_NODES] + partials[NPAD:NPAD + N_NODES])[:, None]


# speedup vs baseline: 1.3912x; 1.3912x over previous
import jax
import jax.numpy as jnp
import numpy as np
from jax import lax
from jax.experimental import pallas as pl
from jax.experimental.pallas import tpu as pltpu
from jax.experimental.pallas import tpu_sc as plsc

N_NODES = 100000
N_EDGES = 6400000
NUM_TYPES = 16
FACTOR = 1.0 / np.sqrt(64.0)

NC = 2
NS = 16
NW = NC * NS
L = 16

K = 2048
NCH = N_EDGES // K
NT_MIN = NCH // NW
NPACK = N_NODES // 4
NT_REM = NCH - NT_MIN * NW
NBUF = 4

NSEG = 6256
NPAD = NS * NSEG


def _sc_body(energy_hbm, eidx_hbm, species_hbm, scale_hbm, out_hbm,
             species_v, scale_v,
             e0, e1, e2, e3, cn0, cn1, cn2, cn3, v0, v1, v2, v3,
             x0, x1, x2, x3, s0, s1, s2, s3, ss0, ss1, zbuf, accum_sh):
    cid = lax.axis_index("c")
    sid = lax.axis_index("s")
    wid = cid * NS + sid

    e_b = (e0, e1, e2, e3)
    cn_b = (cn0, cn1, cn2, cn3)
    v_b = (v0, v1, v2, v3)
    x_b = (x0, x1, x2, x3)
    sem_b = (s0, s1, s2, s3)
    sem_s = (ss0, ss1)

    pltpu.sync_copy(species_hbm, species_v)
    pltpu.sync_copy(scale_hbm, scale_v)

    def zbody(i, _):
        zbuf[pl.ds(i * L, L)] = jnp.zeros((L,), jnp.float32)
        return _

    lax.fori_loop(0, NSEG // L, zbody, None)
    pltpu.sync_copy(zbuf, accum_sh.at[pl.ds(sid * NSEG, NSEG)])
    plsc.subcore_barrier()

    def base_of(t):
        return (t * NW + wid) * K

    def fire_in(t, b):
        base = base_of(t)
        pltpu.async_copy(energy_hbm.at[pl.ds(base, K)], e_b[b], sem_b[b])
        pltpu.async_copy(eidx_hbm.at[:, pl.ds(base, K)], cn_b[b], sem_b[b])

    def wait_in(t, b):
        base = base_of(t)
        pltpu.make_async_copy(energy_hbm.at[pl.ds(base, K)], e_b[b],
                              sem_b[b]).wait()
        pltpu.make_async_copy(eidx_hbm.at[:, pl.ds(base, K)], cn_b[b],
                              sem_b[b]).wait()

    def compute(t, b):
        @plsc.parallel_loop(0, K, step=L, unroll=4)
        def gbody(off):
            ci = cn_b[b][0, pl.ds(off, L)]
            ni = cn_b[b][1, pl.ds(off, L)]
            x_b[b][pl.ds(off, L)] = ci
            wc = plsc.load_gather(species_v, [ci >> 2])
            wn = plsc.load_gather(species_v, [ni >> 2])
            sc = (wc >> ((ci & 3) << 3)) & 0xFF
            sn = (wn >> ((ni & 3) << 3)) & 0xFF
            comb = (sc << 4) + sn
            v_b[b][pl.ds(off, L)] = e_b[b][pl.ds(off, L)] * \
                plsc.load_gather(scale_v, [comb])

    def fire_scatter(b, p):
        pltpu.async_copy(v_b[b], accum_sh.at[x_b[b]], sem_s[p], add=True)

    def wait_scatter(b, p):
        pltpu.make_async_copy(v_b[b], accum_sh.at[x_b[b]], sem_s[p]).wait()

    def step(t, j):
        wait_in(t, j)
        wait_scatter((j + 2) % NBUF, j % 2)
        fire_in(t + 2, (j + 2) % NBUF)
        compute(t, j)
        fire_scatter(j, j % 2)

    fire_in(0, 0)
    fire_in(1, 1)
    for t in range(NBUF):
        wait_in(t, t % NBUF)
        if t >= 2:
            wait_scatter((t + 2) % NBUF, t % 2)
        fire_in(t + 2, (t + 2) % NBUF)
        compute(t, t % NBUF)
        fire_scatter(t % NBUF, t % 2)

    def block_body(t4, _):
        for j in range(NBUF):
            step(t4 * NBUF + j, j)
        return _

    lax.fori_loop(1, (NT_MIN - NBUF) // NBUF, block_body, None)

    extra = wid < NT_REM

    for t in range((NT_MIN - NBUF) // NBUF * NBUF, NT_MIN):
        j = t % NBUF
        wait_in(t, j)
        wait_scatter((j + 2) % NBUF, j % 2)
        if t + 2 < NT_MIN:
            fire_in(t + 2, (j + 2) % NBUF)
        elif t + 2 == NT_MIN:
            @pl.when(extra)
            def _fire_extra(t=t, j=j):
                fire_in(t + 2, (j + 2) % NBUF)
        compute(t, j)
        fire_scatter(j, j % 2)
    for t in (NT_MIN - 2, NT_MIN - 1):
        wait_scatter(t % NBUF, t % 2)

    @pl.when(extra)
    def _extra_chunk():
        j = NT_MIN % NBUF
        wait_in(NT_MIN, j)
        compute(NT_MIN, j)
        pltpu.sync_copy(v_b[j], accum_sh.at[x_b[j]], add=True)

    plsc.subcore_barrier()

    pltpu.sync_copy(accum_sh.at[pl.ds(sid * NSEG, NSEG)], zbuf)
    pltpu.sync_copy(zbuf, out_hbm.at[pl.ds(cid * NPAD + sid * NSEG, NSEG)])


@jax.jit
def _sc_call(energy, eidx, species, scale):
    mesh = plsc.VectorSubcoreMesh(core_axis_name="c", subcore_axis_name="s")
    return pl.kernel(
        _sc_body,
        out_type=jax.ShapeDtypeStruct((NC * NPAD,), jnp.float32),
        mesh=mesh,
        compiler_params=pltpu.CompilerParams(needs_layout_passes=False),
        scratch_types=[
            pltpu.VMEM((NPACK,), jnp.int32),
            pltpu.VMEM((NUM_TYPES * NUM_TYPES,), jnp.float32),
            *([pltpu.VMEM((K,), jnp.float32)] * NBUF),
            *([pltpu.VMEM((2, K), jnp.int32)] * NBUF),
            *([pltpu.VMEM((K,), jnp.float32)] * NBUF),
            *([pltpu.VMEM((K,), jnp.int32)] * NBUF),
            *([pltpu.SemaphoreType.DMA] * NBUF),
            pltpu.SemaphoreType.DMA,
            pltpu.SemaphoreType.DMA,
            pltpu.VMEM((NSEG,), jnp.float32),
            pltpu.VMEM_SHARED((NPAD,), jnp.float32),
        ],
    )(energy, eidx, species, scale)


def kernel(edge_energy, per_edge_scales, edge_index, atom_types):
    energy = edge_energy.reshape(N_EDGES)
    scale = (per_edge_scales * FACTOR).reshape(NUM_TYPES * NUM_TYPES)
    sp8 = atom_types.reshape(N_NODES).astype(jnp.int8)
    species = jax.lax.bitcast_convert_type(sp8.reshape(NPACK, 4), jnp.int32)
    partials = _sc_call(energy, edge_index, species, scale)
    return (partials[:N_NODES] + partials[NPAD:NPAD + N_NODES])[:, None]
